# Initial kernel scaffold; baseline (speedup 1.0000x reference)
import jax
import jax.numpy as jnp
from jax.experimental import pallas as pl

N = 4096
D = 512
KP1 = 11
BM = 256
NB = N // BM


def _pbe_body(xr_ref, xt_ref, out_ref):
    xr = xr_ref[...]
    xt = xt_ref[...]
    dot = jax.lax.dot_general(
        xr, xt, (((1,), (0,)), ((), ())),
        preferred_element_type=jnp.float32)
    xrf = xr.astype(jnp.float32)
    xtf = xt.astype(jnp.float32)
    sq_r = jnp.sum(xrf * xrf, axis=1)
    sq_a = jnp.sum(xtf * xtf, axis=0)
    d2 = jnp.maximum(sq_r[:, None] + sq_a[None, :] - 2.0 * dot, 0.0)

    inf = jnp.float32(jnp.inf)
    s = jnp.zeros((BM,), jnp.float32)
    rem = jnp.full((BM,), float(KP1), jnp.float32)
    m0 = None
    for t in range(KP1):
        m = jnp.min(d2, axis=1)
        if t == 0:
            m0 = m
        c = d2 <= m[:, None]
        cnt = jnp.sum(c.astype(jnp.float32), axis=1)
        take = jnp.minimum(cnt, rem)
        s = s + jnp.where(take > 0.0, take * jnp.sqrt(m), 0.0)
        rem = rem - take
        d2 = jnp.where(c, inf, d2)
    rew = jnp.log1p((s - jnp.sqrt(m0)) / (KP1 - 1))
    out_ref[0, :] = rew


@jax.jit
def kernel(obs):
    xb = obs.astype(jnp.bfloat16)
    out = pl.pallas_call(
        _pbe_body,
        grid=(NB,),
        in_specs=[
            pl.BlockSpec((BM, D), lambda i: (i, 0)),
            pl.BlockSpec((D, N), lambda i: (0, 0)),
        ],
        out_specs=pl.BlockSpec((1, BM), lambda i: (i, 0)),
        out_shape=jax.ShapeDtypeStruct((NB, BM), jnp.float32),
    )(xb, xb.T)
    return out.reshape(N)


# speedup vs baseline: 12.5357x; 12.5357x over previous
import jax
import jax.numpy as jnp
from jax.experimental import pallas as pl

N = 4096
D = 512
KP1 = 11
BM = 256
NB = N // BM


def _pbe_body(xr_ref, xt_ref, out_ref):
    xr = xr_ref[...]
    xt = xt_ref[...]
    dot = jax.lax.dot_general(
        xr, xt, (((1,), (0,)), ((), ())),
        preferred_element_type=jnp.float32)
    xrf = xr.astype(jnp.float32)
    xtf = xt.astype(jnp.float32)
    sq_r = jnp.sum(xrf * xrf, axis=1)
    sq_a = jnp.sum(xtf * xtf, axis=0)
    d2 = jnp.maximum(sq_r[:, None] + sq_a[None, :] - 2.0 * dot, 0.0)

    inf = jnp.float32(jnp.inf)
    s = jnp.zeros((BM,), jnp.float32)
    rem = jnp.full((BM,), float(KP1), jnp.float32)
    m0 = None
    for t in range(KP1):
        m = jnp.min(d2, axis=1)
        if t == 0:
            m0 = m
        c = d2 <= m[:, None]
        cnt = jnp.sum(c.astype(jnp.float32), axis=1)
        take = jnp.minimum(cnt, rem)
        s = s + jnp.where(take > 0.0, take * jnp.sqrt(m), 0.0)
        rem = rem - take
        d2 = jnp.where(c, inf, d2)
    rew = jnp.log1p((s - jnp.sqrt(m0)) / (KP1 - 1))
    out_ref[0, 0, :] = rew


@jax.jit
def kernel(obs):
    xb = obs.astype(jnp.bfloat16)
    out = pl.pallas_call(
        _pbe_body,
        grid=(NB,),
        in_specs=[
            pl.BlockSpec((BM, D), lambda i: (i, 0)),
            pl.BlockSpec((D, N), lambda i: (0, 0)),
        ],
        out_specs=pl.BlockSpec((1, 1, BM), lambda i: (i, 0, 0)),
        out_shape=jax.ShapeDtypeStruct((NB, 1, BM), jnp.float32),
    )(xb, xb.T)
    return out.reshape(N)


# speedup vs baseline: 27.4152x; 2.1870x over previous
import jax
import jax.numpy as jnp
from jax.experimental import pallas as pl

N = 4096
D = 512
KP1 = 11
BM = 256
NB = N // BM


def _pbe_body(xr_ref, xt_ref, out_ref):
    xr = xr_ref[...]
    xt = xt_ref[...]
    dot = jax.lax.dot_general(
        xr, xt, (((1,), (0,)), ((), ())),
        preferred_element_type=jnp.float32)
    xrf = xr.astype(jnp.float32)
    xtf = xt.astype(jnp.float32)
    sq_r = jnp.sum(xrf * xrf, axis=1)
    sq_a = jnp.sum(xtf * xtf, axis=0)
    d2 = jnp.maximum(sq_r[:, None] + sq_a[None, :] - 2.0 * dot, 0.0)

    inf = jnp.float32(jnp.inf)
    t1 = jnp.full((BM, 128), inf, jnp.float32)
    t2 = jnp.full((BM, 128), inf, jnp.float32)
    t3 = jnp.full((BM, 128), inf, jnp.float32)
    t4 = jnp.full((BM, 128), inf, jnp.float32)
    for j in range(N // 128):
        v = d2[:, j * 128:(j + 1) * 128]
        h = jnp.maximum(t1, v)
        t1 = jnp.minimum(t1, v)
        h2 = jnp.maximum(t2, h)
        t2 = jnp.minimum(t2, h)
        h3 = jnp.maximum(t3, h2)
        t3 = jnp.minimum(t3, h2)
        t4 = jnp.minimum(t4, h3)
    cand = jnp.concatenate([t1, t2, t3, t4], axis=1)

    s = jnp.zeros((BM,), jnp.float32)
    rem = jnp.full((BM,), float(KP1), jnp.float32)
    m0 = None
    for t in range(KP1):
        m = jnp.min(cand, axis=1)
        if t == 0:
            m0 = m
        c = cand <= m[:, None]
        cnt = jnp.sum(c.astype(jnp.float32), axis=1)
        take = jnp.minimum(cnt, rem)
        s = s + jnp.where(take > 0.0, take * jnp.sqrt(m), 0.0)
        rem = rem - take
        cand = jnp.where(c, inf, cand)
    rew = jnp.log1p((s - jnp.sqrt(m0)) / (KP1 - 1))
    out_ref[0, 0, :] = rew


@jax.jit
def kernel(obs):
    xb = obs.astype(jnp.bfloat16)
    out = pl.pallas_call(
        _pbe_body,
        grid=(NB,),
        in_specs=[
            pl.BlockSpec((BM, D), lambda i: (i, 0)),
            pl.BlockSpec((D, N), lambda i: (0, 0)),
        ],
        out_specs=pl.BlockSpec((1, 1, BM), lambda i: (i, 0, 0)),
        out_shape=jax.ShapeDtypeStruct((NB, 1, BM), jnp.float32),
    )(xb, xb.T)
    return out.reshape(N)


# speedup vs baseline: 33.5695x; 1.2245x over previous
import jax
import jax.numpy as jnp
from jax.experimental import pallas as pl
from jax.experimental.pallas import tpu as pltpu

N = 4096
D = 512
KP1 = 11
BM = 512
NB = N // BM
CW = 512
NC = N // CW


def _pbe_body(xr_ref, xt_ref, out_ref, sqa_ref):
    i = pl.program_id(0)
    xr = xr_ref[...]
    xrf = xr.astype(jnp.float32)
    sq_r = jnp.sum(xrf * xrf, axis=1)

    @pl.when(i == 0)
    def _():
        xtf = xt_ref[...].astype(jnp.float32)
        sqa_ref[0, :] = jnp.sum(xtf * xtf, axis=0)

    inf = jnp.float32(jnp.inf)
    t1 = jnp.full((BM, 128), inf, jnp.float32)
    t2 = jnp.full((BM, 128), inf, jnp.float32)
    t3 = jnp.full((BM, 128), inf, jnp.float32)
    for c in range(NC):
        xtc = xt_ref[:, c * CW:(c + 1) * CW]
        dotc = jax.lax.dot_general(
            xr, xtc, (((1,), (0,)), ((), ())),
            preferred_element_type=jnp.float32)
        sq_c = sqa_ref[0, c * CW:(c + 1) * CW]
        d2c = jnp.maximum(sq_r[:, None] + sq_c[None, :] - 2.0 * dotc, 0.0)
        for j in range(CW // 128):
            v = d2c[:, j * 128:(j + 1) * 128]
            h = jnp.maximum(t1, v)
            t1 = jnp.minimum(t1, v)
            h2 = jnp.maximum(t2, h)
            t2 = jnp.minimum(t2, h)
            t3 = jnp.minimum(t3, h2)
    cand = jnp.concatenate([t1, t2, t3], axis=1)

    s = jnp.zeros((BM,), jnp.float32)
    rem = jnp.full((BM,), float(KP1), jnp.float32)
    m0 = None
    for t in range(KP1):
        m = jnp.min(cand, axis=1)
        if t == 0:
            m0 = m
        c = cand <= m[:, None]
        cnt = jnp.sum(c.astype(jnp.float32), axis=1)
        take = jnp.minimum(cnt, rem)
        s = s + jnp.where(take > 0.0, take * jnp.sqrt(m), 0.0)
        rem = rem - take
        cand = jnp.where(c, inf, cand)
    rew = jnp.log1p((s - jnp.sqrt(m0)) / (KP1 - 1))
    out_ref[0, 0, :] = rew


@jax.jit
def kernel(obs):
    xb = obs.astype(jnp.bfloat16)
    out = pl.pallas_call(
        _pbe_body,
        grid=(NB,),
        in_specs=[
            pl.BlockSpec((BM, D), lambda i: (i, 0)),
            pl.BlockSpec((D, N), lambda i: (0, 0)),
        ],
        out_specs=pl.BlockSpec((1, 1, BM), lambda i: (i, 0, 0)),
        out_shape=jax.ShapeDtypeStruct((NB, 1, BM), jnp.float32),
        scratch_shapes=[pltpu.VMEM((1, N), jnp.float32)],
    )(xb, xb.T)
    return out.reshape(N)


# speedup vs baseline: 38.5343x; 1.1479x over previous
import jax
import jax.numpy as jnp
from jax.experimental import pallas as pl
from jax.experimental.pallas import tpu as pltpu

N = 4096
D = 512
KP1 = 11
BM = 512
NB = N // BM
CW = 512
NC = N // CW


def _pbe_body(xr_ref, xt_ref, out_ref, sqa_ref):
    i = pl.program_id(0)
    xr = xr_ref[...]
    xrf = xr.astype(jnp.float32)
    sq_r = jnp.sum(xrf * xrf, axis=1)

    @pl.when(i == 0)
    def _():
        xtf = xt_ref[...].astype(jnp.float32)
        sqa_ref[0, :] = jnp.sum(xtf * xtf, axis=0)

    inf = jnp.float32(jnp.inf)
    xr2 = xr * jnp.bfloat16(-2.0)
    t1 = jnp.full((BM, 128), inf, jnp.float32)
    t2 = jnp.full((BM, 128), inf, jnp.float32)
    t3 = jnp.full((BM, 128), inf, jnp.float32)
    for c in range(NC):
        xtc = xt_ref[:, c * CW:(c + 1) * CW]
        dotc = jax.lax.dot_general(
            xr2, xtc, (((1,), (0,)), ((), ())),
            preferred_element_type=jnp.float32)
        sq_c = sqa_ref[0, c * CW:(c + 1) * CW]
        d2c = (sq_r[:, None] + sq_c[None, :]) + dotc
        for j in range(CW // 128):
            v = d2c[:, j * 128:(j + 1) * 128]
            h = jnp.maximum(t1, v)
            t1 = jnp.minimum(t1, v)
            h2 = jnp.maximum(t2, h)
            t2 = jnp.minimum(t2, h)
            t3 = jnp.minimum(t3, h2)

    zero = jnp.float32(0.0)
    s = jnp.zeros((BM,), jnp.float32)
    rem = jnp.full((BM,), float(KP1), jnp.float32)
    m0 = None
    for t in range(KP1):
        m = jnp.min(t1, axis=1)
        r = jnp.sqrt(jnp.maximum(m, zero))
        if t == 0:
            m0r = r
        c = t1 <= m[:, None]
        cnt = jnp.sum(c.astype(jnp.float32), axis=1)
        take = jnp.minimum(cnt, rem)
        s = s + jnp.where(take > 0.0, take * r, 0.0)
        rem = rem - take
        t1 = jnp.where(c, t2, t1)
        t2 = jnp.where(c, t3, t2)
        t3 = jnp.where(c, inf, t3)
    rew = jnp.log1p((s - m0r) / (KP1 - 1))
    out_ref[0, 0, :] = rew


@jax.jit
def kernel(obs):
    xb = obs.astype(jnp.bfloat16)
    out = pl.pallas_call(
        _pbe_body,
        grid=(NB,),
        in_specs=[
            pl.BlockSpec((BM, D), lambda i: (i, 0)),
            pl.BlockSpec((D, N), lambda i: (0, 0)),
        ],
        out_specs=pl.BlockSpec((1, 1, BM), lambda i: (i, 0, 0)),
        out_shape=jax.ShapeDtypeStruct((NB, 1, BM), jnp.float32),
        scratch_shapes=[pltpu.VMEM((1, N), jnp.float32)],
    )(xb, xb.T)
    return out.reshape(N)
